# Initial kernel scaffold; baseline (speedup 1.0000x reference)
import jax
import jax.numpy as jnp
from jax.experimental import pallas as pl


def kernel(x, lifecycle_states, blueprint_ids, grafting_strategies, blueprint_weights):
    raise NotImplementedError("write your pallas kernel here")


# speedup vs baseline: 2.8335x; 2.8335x over previous
import functools

import jax
import jax.numpy as jnp
from jax.experimental import pallas as pl

_NUM_SEEDS = 64
_HIDDEN = 2048
_CHUNK = _HIDDEN // _NUM_SEEDS
_NUM_BP = 10
_BT = 512


def _combine_kernel(lc_ref, bp_ref, st_ref, e_ref, bw_ref, x_ref, o_ref):
    lc = lc_ref[...]
    bp = bp_ref[...]
    st = st_ref[...]

    active = (lc >= 2) & (lc <= 5) & (bp < _NUM_BP)
    act0 = active & (st == 0)
    act1 = active & (st == 1)
    actm = active & (st >= 2)

    one = jnp.float32(1.0)
    half = jnp.float32(0.5)
    m1 = jnp.where(act0, 0.0, jnp.where(act1, one, jnp.where(actm, half, one)))
    m2 = jnp.where(act1, one, jnp.where(actm, half, 0.0))
    m3 = jnp.where(act0, one, 0.0)

    bpc = jnp.clip(bp, 0, _NUM_BP - 1)
    rows = [m1.astype(jnp.float32), m2.astype(jnp.float32), m3.astype(jnp.float32)]
    for r in range(_NUM_BP):
        rows.append((bpc == r).astype(jnp.float32))
    p = jnp.concatenate(rows, axis=0)

    q = jnp.dot(p, e_ref[...], preferred_element_type=jnp.float32)
    m1f = q[0:1, :]
    m2f = q[1:2, :]
    m3f = q[2:3, :]
    onehot = q[3:3 + _NUM_BP, :]

    w = jnp.sum(onehot * bw_ref[...], axis=0, keepdims=True)
    a = m1f + w * m3f
    b = w * m2f
    o_ref[...] = x_ref[...] * a + b


@jax.jit
def kernel(x, lifecycle_states, blueprint_ids, grafting_strategies, blueprint_weights):
    tokens = x.shape[0]
    lc = lifecycle_states.reshape(1, _NUM_SEEDS)
    bp = blueprint_ids.reshape(1, _NUM_SEEDS)
    st = grafting_strategies.reshape(1, _NUM_SEEDS)
    e = (jnp.arange(_HIDDEN, dtype=jnp.int32)[None, :] // _CHUNK
         == jnp.arange(_NUM_SEEDS, dtype=jnp.int32)[:, None]).astype(jnp.float32)

    grid = (tokens // _BT,)
    small = lambda i: (0, 0)
    return pl.pallas_call(
        _combine_kernel,
        grid=grid,
        in_specs=[
            pl.BlockSpec((1, _NUM_SEEDS), small),
            pl.BlockSpec((1, _NUM_SEEDS), small),
            pl.BlockSpec((1, _NUM_SEEDS), small),
            pl.BlockSpec((_NUM_SEEDS, _HIDDEN), small),
            pl.BlockSpec((_NUM_BP, _HIDDEN), small),
            pl.BlockSpec((_BT, _HIDDEN), lambda i: (i, 0)),
        ],
        out_specs=pl.BlockSpec((_BT, _HIDDEN), lambda i: (i, 0)),
        out_shape=jax.ShapeDtypeStruct((tokens, _HIDDEN), x.dtype),
    )(lc, bp, st, e, blueprint_weights, x)


# speedup vs baseline: 2.9052x; 1.0253x over previous
import functools

import jax
import jax.numpy as jnp
from jax.experimental import pallas as pl

_NUM_SEEDS = 64
_HIDDEN = 2048
_CHUNK = _HIDDEN // _NUM_SEEDS
_NUM_BP = 10
_BT = 1024


def _combine_kernel(lc_ref, bp_ref, st_ref, e_ref, bw_ref, x_ref, o_ref):
    lc = lc_ref[...]
    bp = bp_ref[...]
    st = st_ref[...]

    active = (lc >= 2) & (lc <= 5) & (bp < _NUM_BP)
    act0 = active & (st == 0)
    act1 = active & (st == 1)
    actm = active & (st >= 2)

    one = jnp.float32(1.0)
    half = jnp.float32(0.5)
    m1 = jnp.where(act0, 0.0, jnp.where(act1, one, jnp.where(actm, half, one)))
    m2 = jnp.where(act1, one, jnp.where(actm, half, 0.0))
    m3 = jnp.where(act0, one, 0.0)

    bpc = jnp.clip(bp, 0, _NUM_BP - 1)
    rows = [m1.astype(jnp.float32), m2.astype(jnp.float32), m3.astype(jnp.float32)]
    for r in range(_NUM_BP):
        rows.append((bpc == r).astype(jnp.float32))
    p = jnp.concatenate(rows, axis=0)

    q = jnp.dot(p, e_ref[...], preferred_element_type=jnp.float32)
    m1f = q[0:1, :]
    m2f = q[1:2, :]
    m3f = q[2:3, :]
    onehot = q[3:3 + _NUM_BP, :]

    w = jnp.sum(onehot * bw_ref[...], axis=0, keepdims=True)
    a = m1f + w * m3f
    b = w * m2f
    o_ref[...] = x_ref[...] * a + b


@jax.jit
def kernel(x, lifecycle_states, blueprint_ids, grafting_strategies, blueprint_weights):
    tokens = x.shape[0]
    lc = lifecycle_states.reshape(1, _NUM_SEEDS)
    bp = blueprint_ids.reshape(1, _NUM_SEEDS)
    st = grafting_strategies.reshape(1, _NUM_SEEDS)
    e = (jnp.arange(_HIDDEN, dtype=jnp.int32)[None, :] // _CHUNK
         == jnp.arange(_NUM_SEEDS, dtype=jnp.int32)[:, None]).astype(jnp.float32)

    grid = (tokens // _BT,)
    small = lambda i: (0, 0)
    return pl.pallas_call(
        _combine_kernel,
        grid=grid,
        in_specs=[
            pl.BlockSpec((1, _NUM_SEEDS), small),
            pl.BlockSpec((1, _NUM_SEEDS), small),
            pl.BlockSpec((1, _NUM_SEEDS), small),
            pl.BlockSpec((_NUM_SEEDS, _HIDDEN), small),
            pl.BlockSpec((_NUM_BP, _HIDDEN), small),
            pl.BlockSpec((_BT, _HIDDEN), lambda i: (i, 0)),
        ],
        out_specs=pl.BlockSpec((_BT, _HIDDEN), lambda i: (i, 0)),
        out_shape=jax.ShapeDtypeStruct((tokens, _HIDDEN), x.dtype),
    )(lc, bp, st, e, blueprint_weights, x)
